# Initial kernel scaffold; baseline (speedup 1.0000x reference)
import jax
import jax.numpy as jnp
from jax.experimental import pallas as pl


def kernel(x0, x1, edge_index0, edge_index1, batch0, batch1, Wr, br, Wg, bg, ln_g, ln_b, Wc, bc):
    raise NotImplementedError("write your pallas kernel here")


# speedup vs baseline: 6.0672x; 6.0672x over previous
import jax
import jax.numpy as jnp
from jax import lax
from jax.experimental import pallas as pl
from jax.experimental.pallas import tpu as pltpu, tpu_sc as plsc

_NC = 2
_NS = 16
_K = 80
_ZR = 32
_F32 = jnp.float32


def _sc_mesh():
    return plsc.VectorSubcoreMesh(
        core_axis_name="c", subcore_axis_name="s",
        num_cores=_NC, num_subcores=_NS)


def _sc_deg(dst_chunk, n, per_tile, acc_rows):
    nchunk = per_tile // _K
    rows_out = acc_rows // _NS
    zcopies = rows_out // _ZR

    def body(dst_hbm, deg_hbm, dstbuf, ones_v, zbuf, acc):
        c = lax.axis_index("c")
        s = lax.axis_index("s")
        z16 = jnp.zeros((16,), _F32)
        o16 = jnp.ones((16,), _F32)

        def fill(i, carry):
            zbuf[i, pl.ds(0, 16)] = z16
            return carry
        lax.fori_loop(0, _ZR, fill, 0)

        def fill1(i, carry):
            ones_v[i, pl.ds(0, 16)] = o16
            return carry
        lax.fori_loop(0, _K, fill1, 0)

        for rz in range(zcopies):
            pltpu.sync_copy(zbuf, acc.at[pl.ds((s * zcopies + rz) * _ZR, _ZR)])
        plsc.subcore_barrier()

        pltpu.sync_copy(dst_hbm.at[c, s], dstbuf)

        def ch(j, carry):
            pltpu.sync_copy(ones_v, acc.at[dstbuf.at[j]], add=True)
            return carry
        lax.fori_loop(0, nchunk, ch, 0)
        plsc.subcore_barrier()

        row0 = s * rows_out
        pltpu.sync_copy(acc.at[pl.ds(row0, rows_out)],
                        deg_hbm.at[c, pl.ds(row0, rows_out)])

    return pl.kernel(
        body,
        out_type=jax.ShapeDtypeStruct((2, acc_rows, 16), _F32),
        mesh=_sc_mesh(),
        scratch_types=[
            pltpu.VMEM((nchunk, _K), jnp.int32),
            pltpu.VMEM((_K, 16), _F32),
            pltpu.VMEM((_ZR, 16), _F32),
            pltpu.VMEM_SHARED((acc_rows, 16), _F32),
        ],
    )(dst_chunk)


def _sc_agg(tab, src_flat, dst_chunk, n, per_tile, acc_rows, dh):
    nchunk = per_tile // _K
    rows_io = acc_rows // _NS
    zcopies = rows_io // _ZR

    def body(tab_hbm, src_hbm, dst_hbm, agg_hbm,
             srcbuf, dstbuf, gbuf, zbuf, acc, sem):
        c = lax.axis_index("c")
        s = lax.axis_index("s")
        z16 = jnp.zeros((16,), _F32)
        ncol = dh // 16

        def zb(i, carry):
            zbuf[i // ncol, pl.ds((i % ncol) * 16, 16)] = z16
            return carry
        lax.fori_loop(0, _ZR * ncol, zb, 0)

        for g in range(2):
            for r in range(zcopies):
                pltpu.sync_copy(
                    zbuf, acc.at[pl.ds((s * zcopies + r) * _ZR, _ZR)])
            plsc.subcore_barrier()

            pltpu.sync_copy(src_hbm.at[g, s], srcbuf)
            pltpu.sync_copy(dst_hbm.at[g, s], dstbuf)
            off = jnp.broadcast_to(c * (2 * n) + g * n, (16,)).astype(jnp.int32)

            def ob(i, carry):
                srcbuf[pl.ds(i * 16, 16)] = srcbuf[pl.ds(i * 16, 16)] + off
                return carry
            lax.fori_loop(0, per_tile // 16, ob, 0)

            def ch(j, carry):
                pltpu.async_copy(
                    tab_hbm.at[srcbuf.at[pl.ds(j * _K, _K)]], gbuf, sem
                ).wait()
                pltpu.sync_copy(gbuf, acc.at[dstbuf.at[j]], add=True)
                return carry
            lax.fori_loop(0, nchunk, ch, 0)

            plsc.subcore_barrier()
            out0 = (g * 2 + c) * acc_rows + s * rows_io
            pltpu.sync_copy(acc.at[pl.ds(s * rows_io, rows_io)],
                            agg_hbm.at[pl.ds(out0, rows_io)])
            plsc.subcore_barrier()

    return pl.kernel(
        body,
        out_type=jax.ShapeDtypeStruct((4 * acc_rows, dh), _F32),
        mesh=_sc_mesh(),
        scratch_types=[
            pltpu.VMEM((per_tile,), jnp.int32),
            pltpu.VMEM((nchunk, _K), jnp.int32),
            pltpu.VMEM((_K, dh), _F32),
            pltpu.VMEM((_ZR, dh), _F32),
            pltpu.VMEM_SHARED((acc_rows, dh), _F32),
            pltpu.SemaphoreType.DMA,
        ],
    )(tab, src_flat, dst_chunk)


def _tc_prologue(xs, deg, Wr, br2, Wg0, r):
    g_, n, d = xs.shape
    nb = n // r
    dh = d // 2

    def body(x_ref, deg_ref, wr_ref, br_ref, wg0_ref, xo_ref, dv_ref, yw_ref):
        xw = jnp.dot(x_ref[0], wr_ref[0], preferred_element_type=_F32) + br_ref[0]
        dinv = lax.rsqrt(deg_ref[0])
        y = jnp.dot(dinv * xw, wg0_ref[0], preferred_element_type=_F32)
        xo_ref[0] = xw
        dv_ref[0] = dinv
        yw_ref[0, 0] = y[:, :dh]
        yw_ref[1, 0] = y[:, dh:]

    return pl.pallas_call(
        body,
        grid=(g_, nb),
        in_specs=[
            pl.BlockSpec((1, r, d), lambda g, i: (g, i, 0)),
            pl.BlockSpec((1, r, 1), lambda g, i: (g, i, 0)),
            pl.BlockSpec((1, d, d), lambda g, i: (g, 0, 0)),
            pl.BlockSpec((1, 1, d), lambda g, i: (g, 0, 0)),
            pl.BlockSpec((1, d, d), lambda g, i: (g, 0, 0)),
        ],
        out_specs=[
            pl.BlockSpec((1, r, d), lambda g, i: (g, i, 0)),
            pl.BlockSpec((1, r, 1), lambda g, i: (g, i, 0)),
            pl.BlockSpec((2, 1, r, dh), lambda g, i: (0, g, i, 0)),
        ],
        out_shape=[
            jax.ShapeDtypeStruct((g_, n, d), _F32),
            jax.ShapeDtypeStruct((g_, n, 1), _F32),
            jax.ShapeDtypeStruct((2, g_, n, dh), _F32),
        ],
    )(xs, deg, Wr, br2, Wg0)


def _tc_layer(x, agg, dinv, bg_l, lng_l, lnb_l, Wg_next, r):
    g_, n, d = x.shape
    nb = n // r
    dh = d // 2
    last = Wg_next is None

    def body(x_ref, a_ref, dv_ref, bg_ref, lg_ref, lb_ref, *rest):
        if last:
            (xo_ref,) = rest
        else:
            wn_ref, xo_ref, yw_ref = rest
        aggf = jnp.concatenate([a_ref[0, 0], a_ref[0, 1]], axis=-1)
        dinv = dv_ref[0]
        cv = dinv * aggf + bg_ref[0]
        mu = jnp.mean(cv, axis=-1, keepdims=True)
        var = jnp.mean((cv - mu) ** 2, axis=-1, keepdims=True)
        cv = (cv - mu) * lax.rsqrt(var + 1e-5) * lg_ref[0] + lb_ref[0]
        xn = x_ref[0] + jnp.maximum(cv, 0.0)
        xo_ref[0] = xn
        if not last:
            y = jnp.dot(dinv * xn, wn_ref[0], preferred_element_type=_F32)
            yw_ref[0, 0] = y[:, :dh]
            yw_ref[1, 0] = y[:, dh:]

    in_specs = [
        pl.BlockSpec((1, r, d), lambda g, i: (g, i, 0)),
        pl.BlockSpec((1, 2, r, dh), lambda g, i: (g, 0, i, 0)),
        pl.BlockSpec((1, r, 1), lambda g, i: (g, i, 0)),
        pl.BlockSpec((1, 1, d), lambda g, i: (g, 0, 0)),
        pl.BlockSpec((1, 1, d), lambda g, i: (g, 0, 0)),
        pl.BlockSpec((1, 1, d), lambda g, i: (g, 0, 0)),
    ]
    out_specs = [pl.BlockSpec((1, r, d), lambda g, i: (g, i, 0))]
    out_shape = [jax.ShapeDtypeStruct((g_, n, d), _F32)]
    args = [x, agg, dinv, bg_l, lng_l, lnb_l]
    if not last:
        in_specs.append(pl.BlockSpec((1, d, d), lambda g, i: (g, 0, 0)))
        out_specs.append(pl.BlockSpec((2, 1, r, dh), lambda g, i: (0, g, i, 0)))
        out_shape.append(jax.ShapeDtypeStruct((2, g_, n, dh), _F32))
        args.append(Wg_next)

    res = pl.pallas_call(
        body, grid=(g_, nb), in_specs=in_specs,
        out_specs=out_specs, out_shape=out_shape,
    )(*args)
    return (res[0], None) if last else (res[0], res[1])


def _tc_pool(x, batch3, r):
    g_, n, d = x.shape
    nb = n // r

    def body(x_ref, b_ref, f_ref, acc):
        i = pl.program_id(1)

        @pl.when(i == 0)
        def _():
            acc[...] = jnp.zeros_like(acc)

        bt = b_ref[0, 0, 0, :]
        oh = (bt[:, None] == lax.broadcasted_iota(jnp.int32, (r, 8), 1)
              ).astype(_F32)
        xa = jnp.concatenate([x_ref[0], jnp.ones((r, 128), _F32)], axis=-1)
        acc[...] += jnp.dot(oh.T, xa, preferred_element_type=_F32)

        @pl.when(i == nb - 1)
        def _():
            f_ref[0] = acc[:, :d] / jnp.maximum(acc[:, d:d + 1], 1.0)

    return pl.pallas_call(
        body,
        grid=(g_, nb),
        in_specs=[
            pl.BlockSpec((1, r, d), lambda g, i: (g, i, 0)),
            pl.BlockSpec((1, 1, 1, r), lambda g, i: (g, i, 0, 0)),
        ],
        out_specs=pl.BlockSpec((1, 8, d), lambda g, i: (g, 0, 0)),
        out_shape=jax.ShapeDtypeStruct((g_, 8, d), _F32),
        scratch_shapes=[pltpu.VMEM((8, d + 128), _F32)],
    )(x, batch3)


def _tc_logits(feats, Wc2, bc2):
    g_, _, d = feats.shape
    c = Wc2.shape[-1]

    def body(f_ref, wc_ref, bc_ref, o_ref):
        o_ref[...] = (
            jnp.dot(f_ref[0], wc_ref[0], preferred_element_type=_F32)
            + jnp.dot(f_ref[1], wc_ref[1], preferred_element_type=_F32)
            + bc_ref[...])

    return pl.pallas_call(
        body,
        out_shape=jax.ShapeDtypeStruct((8, c), _F32),
    )(feats, Wc2, bc2)


def kernel(x0, x1, edge_index0, edge_index1, batch0, batch1,
           Wr, br, Wg, bg, ln_g, ln_b, Wc, bc):
    n, d = x0.shape
    e = edge_index0.shape[1]
    nlayers = Wg.shape[1]
    ncls = Wc.shape[-1]
    dh = d // 2
    r = 2000
    nb = n // r

    ep = e + n
    per_tile = -(-ep // (_NS * 2 * _K)) * (2 * _K)
    epad = per_tile * _NS
    acc_rows = -(-(n + 1) // (_NS * _ZR)) * (_NS * _ZR)

    ii = jnp.int32
    loop = jnp.arange(n, dtype=ii)
    pads = epad - ep

    def prep(eidx):
        srcv = jnp.concatenate([eidx[0], loop, jnp.zeros((pads,), ii)])
        dstv = jnp.concatenate([eidx[1], loop, jnp.full((pads,), n, ii)])
        return srcv, dstv

    s0, d0 = prep(edge_index0)
    s1, d1 = prep(edge_index1)
    src_flat = jnp.stack([s0, s1]).reshape(2, _NS, per_tile)
    dst_flat = jnp.stack([d0, d1]).reshape(2, _NS, per_tile)
    dst_chunk = dst_flat.reshape(2, _NS, per_tile // _K, _K)

    deg_full = _sc_deg(dst_chunk, n, per_tile, acc_rows)
    deg = deg_full[:, :n, :1]

    xs = jnp.stack([x0, x1])
    x, dinv, yw = _tc_prologue(
        xs, deg, Wr, br.reshape(2, 1, d), Wg[:, 0], r)

    for l in range(nlayers):
        tab = yw.reshape(4 * n, dh)
        agg = _sc_agg(tab, src_flat, dst_chunk, n, per_tile, acc_rows, dh
                      ).reshape(2, 2, acc_rows, dh)
        wn = Wg[:, l + 1] if l + 1 < nlayers else None
        x, yw = _tc_layer(
            x, agg, dinv,
            bg[:, l].reshape(2, 1, d),
            ln_g[:, l].reshape(2, 1, d),
            ln_b[:, l].reshape(2, 1, d),
            wn, r)

    batch3 = jnp.stack([batch0, batch1]).reshape(2, nb, 1, r)
    feats = _tc_pool(x, batch3, r)
    return _tc_logits(feats, Wc.reshape(2, d, ncls), bc.reshape(1, ncls))


# speedup vs baseline: 8.3087x; 1.3694x over previous
import jax
import jax.numpy as jnp
from jax import lax
from jax.experimental import pallas as pl
from jax.experimental.pallas import tpu as pltpu, tpu_sc as plsc

_NC = 2
_NS = 16
_K = 80
_ZR = 8
_F32 = jnp.float32


def _sc_mesh():
    return plsc.VectorSubcoreMesh(
        core_axis_name="c", subcore_axis_name="s",
        num_cores=_NC, num_subcores=_NS)


def _sc_deg(dst_chunk, n, per_tile, acc_rows):
    nchunk = per_tile // _K
    rows_out = acc_rows // _NS
    zcopies = rows_out // _ZR

    def body(dst_hbm, deg_hbm, dstbuf, ones_v, zbuf, acc):
        c = lax.axis_index("c")
        s = lax.axis_index("s")
        z16 = jnp.zeros((16,), _F32)
        o16 = jnp.ones((16,), _F32)

        def fill(i, carry):
            zbuf[i, pl.ds(0, 16)] = z16
            return carry
        lax.fori_loop(0, _ZR, fill, 0)

        def fill1(i, carry):
            ones_v[i, pl.ds(0, 16)] = o16
            return carry
        lax.fori_loop(0, _K, fill1, 0)

        for rz in range(zcopies):
            pltpu.sync_copy(zbuf, acc.at[pl.ds((s * zcopies + rz) * _ZR, _ZR)])
        plsc.subcore_barrier()

        pltpu.sync_copy(dst_hbm.at[c, s], dstbuf)

        def ch(j, carry):
            pltpu.sync_copy(ones_v, acc.at[dstbuf.at[j]], add=True)
            return carry
        lax.fori_loop(0, nchunk, ch, 0)
        plsc.subcore_barrier()

        row0 = s * rows_out
        pltpu.sync_copy(acc.at[pl.ds(row0, rows_out)],
                        deg_hbm.at[c, pl.ds(row0, rows_out)])

    return pl.kernel(
        body,
        out_type=jax.ShapeDtypeStruct((2, acc_rows, 16), _F32),
        mesh=_sc_mesh(),
        scratch_types=[
            pltpu.VMEM((nchunk, _K), jnp.int32),
            pltpu.VMEM((_K, 16), _F32),
            pltpu.VMEM((_ZR, 16), _F32),
            pltpu.VMEM_SHARED((acc_rows, 16), _F32),
        ],
    )(dst_chunk)


def _sc_agg(tab, src_idx, dst_chunk, n, per_tile, acc_rows, dh):
    nchunk = per_tile // _K
    assert nchunk % 2 == 0 and nchunk >= 4
    rows_io = acc_rows // _NS
    zcopies = rows_io // _ZR

    def body(tab_hbm, src_hbm, dst_hbm, z_hbm, agg_hbm,
             srcbuf, dstbuf, ga, gb, acc, sema, semb):
        c = lax.axis_index("c")
        s = lax.axis_index("s")

        def start(j, buf, sem):
            pltpu.async_copy(
                tab_hbm.at[srcbuf.at[pl.ds(j * _K, _K)]], buf, sem)

        def wait(buf, sem):
            pltpu.make_async_copy(tab_hbm.at[pl.ds(0, _K)], buf, sem).wait()

        def scat(j, buf):
            pltpu.sync_copy(buf, acc.at[dstbuf.at[j]], add=True)

        for g in range(2):
            pltpu.sync_copy(z_hbm.at[pl.ds(s * rows_io, rows_io)],
                            acc.at[pl.ds(s * rows_io, rows_io)])
            plsc.subcore_barrier()

            pltpu.sync_copy(src_hbm.at[c, g, s], srcbuf)
            pltpu.sync_copy(dst_hbm.at[g, s], dstbuf)

            start(0, ga, sema)

            def ch(j2, carry):
                j = j2 * 2
                start(j + 1, gb, semb)
                wait(ga, sema)
                scat(j, ga)
                start(j + 2, ga, sema)
                wait(gb, semb)
                scat(j + 1, gb)
                return carry
            lax.fori_loop(0, nchunk // 2 - 1, ch, 0)

            jl = nchunk - 2
            start(jl + 1, gb, semb)
            wait(ga, sema)
            scat(jl, ga)
            wait(gb, semb)
            scat(jl + 1, gb)

            plsc.subcore_barrier()
            out0 = (g * 2 + c) * acc_rows + s * rows_io
            pltpu.sync_copy(acc.at[pl.ds(s * rows_io, rows_io)],
                            agg_hbm.at[pl.ds(out0, rows_io)])
            plsc.subcore_barrier()

    return pl.kernel(
        body,
        out_type=jax.ShapeDtypeStruct((4 * acc_rows, dh), _F32),
        mesh=_sc_mesh(),
        scratch_types=[
            pltpu.VMEM((per_tile,), jnp.int32),
            pltpu.VMEM((nchunk, _K), jnp.int32),
            pltpu.VMEM((_K, dh), _F32),
            pltpu.VMEM((_K, dh), _F32),
            pltpu.VMEM_SHARED((acc_rows, dh), _F32),
            pltpu.SemaphoreType.DMA,
            pltpu.SemaphoreType.DMA,
        ],
    )(tab, src_idx, dst_chunk, jnp.zeros((acc_rows, dh), _F32))


def _tc_prologue(xs, deg, Wr, br2, Wg0, r):
    g_, n, d = xs.shape
    nb = n // r
    dh = d // 2

    def body(x_ref, deg_ref, wr_ref, br_ref, wg0_ref, xo_ref, dv_ref, yw_ref):
        xw = jnp.dot(x_ref[0], wr_ref[0], preferred_element_type=_F32) + br_ref[0]
        dinv = lax.rsqrt(deg_ref[0])
        y = jnp.dot(dinv * xw, wg0_ref[0], preferred_element_type=_F32)
        xo_ref[0] = xw
        dv_ref[0] = dinv
        yw_ref[0, 0] = y[:, :dh]
        yw_ref[1, 0] = y[:, dh:]

    return pl.pallas_call(
        body,
        grid=(g_, nb),
        in_specs=[
            pl.BlockSpec((1, r, d), lambda g, i: (g, i, 0)),
            pl.BlockSpec((1, r, 1), lambda g, i: (g, i, 0)),
            pl.BlockSpec((1, d, d), lambda g, i: (g, 0, 0)),
            pl.BlockSpec((1, 1, d), lambda g, i: (g, 0, 0)),
            pl.BlockSpec((1, d, d), lambda g, i: (g, 0, 0)),
        ],
        out_specs=[
            pl.BlockSpec((1, r, d), lambda g, i: (g, i, 0)),
            pl.BlockSpec((1, r, 1), lambda g, i: (g, i, 0)),
            pl.BlockSpec((2, 1, r, dh), lambda g, i: (0, g, i, 0)),
        ],
        out_shape=[
            jax.ShapeDtypeStruct((g_, n, d), _F32),
            jax.ShapeDtypeStruct((g_, n, 1), _F32),
            jax.ShapeDtypeStruct((2, g_, n, dh), _F32),
        ],
    )(xs, deg, Wr, br2, Wg0)


def _tc_layer(x, agg, dinv, bg_l, lng_l, lnb_l, Wg_next, r):
    g_, n, d = x.shape
    nb = n // r
    dh = d // 2
    last = Wg_next is None

    def body(x_ref, a_ref, dv_ref, bg_ref, lg_ref, lb_ref, *rest):
        if last:
            (xo_ref,) = rest
        else:
            wn_ref, xo_ref, yw_ref = rest
        aggf = jnp.concatenate([a_ref[0, 0], a_ref[0, 1]], axis=-1)
        dinv = dv_ref[0]
        cv = dinv * aggf + bg_ref[0]
        mu = jnp.mean(cv, axis=-1, keepdims=True)
        var = jnp.mean((cv - mu) ** 2, axis=-1, keepdims=True)
        cv = (cv - mu) * lax.rsqrt(var + 1e-5) * lg_ref[0] + lb_ref[0]
        xn = x_ref[0] + jnp.maximum(cv, 0.0)
        xo_ref[0] = xn
        if not last:
            y = jnp.dot(dinv * xn, wn_ref[0], preferred_element_type=_F32)
            yw_ref[0, 0] = y[:, :dh]
            yw_ref[1, 0] = y[:, dh:]

    in_specs = [
        pl.BlockSpec((1, r, d), lambda g, i: (g, i, 0)),
        pl.BlockSpec((1, 2, r, dh), lambda g, i: (g, 0, i, 0)),
        pl.BlockSpec((1, r, 1), lambda g, i: (g, i, 0)),
        pl.BlockSpec((1, 1, d), lambda g, i: (g, 0, 0)),
        pl.BlockSpec((1, 1, d), lambda g, i: (g, 0, 0)),
        pl.BlockSpec((1, 1, d), lambda g, i: (g, 0, 0)),
    ]
    out_specs = [pl.BlockSpec((1, r, d), lambda g, i: (g, i, 0))]
    out_shape = [jax.ShapeDtypeStruct((g_, n, d), _F32)]
    args = [x, agg, dinv, bg_l, lng_l, lnb_l]
    if not last:
        in_specs.append(pl.BlockSpec((1, d, d), lambda g, i: (g, 0, 0)))
        out_specs.append(pl.BlockSpec((2, 1, r, dh), lambda g, i: (0, g, i, 0)))
        out_shape.append(jax.ShapeDtypeStruct((2, g_, n, dh), _F32))
        args.append(Wg_next)

    res = pl.pallas_call(
        body, grid=(g_, nb), in_specs=in_specs,
        out_specs=out_specs, out_shape=out_shape,
    )(*args)
    return (res[0], None) if last else (res[0], res[1])


def _tc_pool(x, batch3, r):
    g_, n, d = x.shape
    nb = n // r

    def body(x_ref, b_ref, f_ref, acc):
        i = pl.program_id(1)

        @pl.when(i == 0)
        def _():
            acc[...] = jnp.zeros_like(acc)

        bt = b_ref[0, 0, 0, :]
        oh = (bt[:, None] == lax.broadcasted_iota(jnp.int32, (r, 8), 1)
              ).astype(_F32)
        xa = jnp.concatenate([x_ref[0], jnp.ones((r, 128), _F32)], axis=-1)
        acc[...] += jnp.dot(oh.T, xa, preferred_element_type=_F32)

        @pl.when(i == nb - 1)
        def _():
            f_ref[0] = acc[:, :d] / jnp.maximum(acc[:, d:d + 1], 1.0)

    return pl.pallas_call(
        body,
        grid=(g_, nb),
        in_specs=[
            pl.BlockSpec((1, r, d), lambda g, i: (g, i, 0)),
            pl.BlockSpec((1, 1, 1, r), lambda g, i: (g, i, 0, 0)),
        ],
        out_specs=pl.BlockSpec((1, 8, d), lambda g, i: (g, 0, 0)),
        out_shape=jax.ShapeDtypeStruct((g_, 8, d), _F32),
        scratch_shapes=[pltpu.VMEM((8, d + 128), _F32)],
    )(x, batch3)


def _tc_logits(feats, Wc2, bc2):
    g_, _, d = feats.shape
    c = Wc2.shape[-1]

    def body(f_ref, wc_ref, bc_ref, o_ref):
        o_ref[...] = (
            jnp.dot(f_ref[0], wc_ref[0], preferred_element_type=_F32)
            + jnp.dot(f_ref[1], wc_ref[1], preferred_element_type=_F32)
            + bc_ref[...])

    return pl.pallas_call(
        body,
        out_shape=jax.ShapeDtypeStruct((8, c), _F32),
    )(feats, Wc2, bc2)


def kernel(x0, x1, edge_index0, edge_index1, batch0, batch1,
           Wr, br, Wg, bg, ln_g, ln_b, Wc, bc):
    n, d = x0.shape
    e = edge_index0.shape[1]
    nlayers = Wg.shape[1]
    ncls = Wc.shape[-1]
    dh = d // 2
    r = 2000
    nb = n // r

    ep = e + n
    per_tile = -(-ep // (_NS * 2 * _K)) * (2 * _K)
    epad = per_tile * _NS
    acc_rows = -(-(n + 1) // (_NS * _ZR)) * (_NS * _ZR)

    ii = jnp.int32
    loop = jnp.arange(n, dtype=ii)
    pads = epad - ep

    def prep(eidx):
        srcv = jnp.concatenate([eidx[0], loop, jnp.zeros((pads,), ii)])
        dstv = jnp.concatenate([eidx[1], loop, jnp.full((pads,), n, ii)])
        return srcv, dstv

    s0, d0 = prep(edge_index0)
    s1, d1 = prep(edge_index1)
    src2 = jnp.stack([s0, s1])
    coff = jnp.arange(2, dtype=ii) * (2 * n)
    goff = jnp.arange(2, dtype=ii) * n
    src_idx = (src2[None] + coff[:, None, None] + goff[None, :, None]
               ).reshape(2, 2, _NS, per_tile)
    dst_flat = jnp.stack([d0, d1]).reshape(2, _NS, per_tile)
    dst_chunk = dst_flat.reshape(2, _NS, per_tile // _K, _K)

    deg_full = _sc_deg(dst_chunk, n, per_tile, acc_rows)
    deg = deg_full[:, :n, :1]

    xs = jnp.stack([x0, x1])
    x, dinv, yw = _tc_prologue(
        xs, deg, Wr, br.reshape(2, 1, d), Wg[:, 0], r)

    for l in range(nlayers):
        tab = yw.reshape(4 * n, dh)
        agg = _sc_agg(tab, src_idx, dst_chunk, n, per_tile, acc_rows, dh
                      ).reshape(2, 2, acc_rows, dh)
        wn = Wg[:, l + 1] if l + 1 < nlayers else None
        x, yw = _tc_layer(
            x, agg, dinv,
            bg[:, l].reshape(2, 1, d),
            ln_g[:, l].reshape(2, 1, d),
            ln_b[:, l].reshape(2, 1, d),
            wn, r)

    batch3 = jnp.stack([batch0, batch1]).reshape(2, nb, 1, r)
    feats = _tc_pool(x, batch3, r)
    return _tc_logits(feats, Wc.reshape(2, d, ncls), bc.reshape(1, ncls))
